# baseline (speedup 1.0000x reference)
import functools
import itertools

import numpy as np
import jax
import jax.numpy as jnp
from jax import lax
from jax.experimental import pallas as pl
from jax.experimental.pallas import tpu as pltpu

_IN_DIM = 10
_NCH = 2
_CONST = 0.8
_NP = 45
_NC = 630


def _pair_combos(n, k):
    x = list(set(frozenset(i) for i in itertools.product(range(n), repeat=k)
                 if len(set(i)) == k))
    return np.array(sorted([sorted(list(i)) for i in x]), dtype=np.int64)


def _disjoint_combos(drcombos):
    combos = []
    dc = [set(i) for i in drcombos.tolist()]
    for idx, i in enumerate(dc):
        for jdx, j in enumerate(dc):
            if not i.intersection(j):
                if [idx, jdx] not in combos and [jdx, idx] not in combos:
                    combos.append([idx, jdx])
    return np.array(sorted(combos), dtype=np.int64)


_DRC = _pair_combos(_IN_DIM, _NCH)
_DRSC = _disjoint_combos(_DRC)

_MPXYZ = np.zeros((4 * _IN_DIM, 48), dtype=np.float32)
for _j in range(_IN_DIM):
    for _k, _comp in enumerate((1, 2, 3)):
        _MPXYZ[4 * _j + _comp, 16 * _k + _j] = 1.0

_DMAT = np.zeros((_IN_DIM, _NP), dtype=np.float32)
for _c, (_i, _j) in enumerate(_DRC):
    _DMAT[_i, _c] = 1.0
    _DMAT[_j, _c] = -1.0

_PS = np.zeros((_NP, _NC), dtype=np.float32)
for _c, (_i, _j) in enumerate(_DRSC):
    _PS[_i, _c] = 1.0
    _PS[_j, _c] = 1.0

_PAYLOAD = np.zeros((_NC, 8), dtype=np.float32)
_PAYLOAD[:, 0:4] = _DRC[_DRSC].reshape(_NC, 4).astype(np.float32)
_PAYLOAD[:, 4] = _DRSC[:, 0].astype(np.float32)
_PAYLOAD[:, 5] = _DRSC[:, 1].astype(np.float32)

_IP0 = _DRC[:, 0].astype(np.int32).reshape(1, _NP)
_IP1 = _DRC[:, 1].astype(np.int32).reshape(1, _NP)
_IC0 = _DRSC[:, 0].astype(np.int32).reshape(1, _NC)
_IC1 = _DRSC[:, 1].astype(np.int32).reshape(1, _NC)

_LIDX = np.arange(_NC, dtype=np.float32).reshape(1, _NC)

_BB = 2048


def _dot(a, b):
    return jnp.dot(a, b, precision=lax.Precision.HIGHEST,
                   preferred_element_type=jnp.float32)


def _asinh(t):
    u = jnp.abs(t)
    r = jnp.log1p(u + u * u / (1.0 + jnp.sqrt(u * u + 1.0)))
    r = jnp.where(u > 1e19, jnp.log(u) + 0.6931471805599453, r)
    return jnp.sign(t) * r


def _gather_lanes(src, idx_row, bb):
    idx = jnp.broadcast_to(idx_row, (bb, idx_row.shape[1]))
    return jnp.take_along_axis(src, idx, axis=1)


def _body(x_ref, mpxyz_ref, p0_ref, p1_ref, c0_ref, c1_ref, li_ref,
          pay_ref, o_ref):
    xr = x_ref[...]
    bb = xr.shape[0]
    pxyz = jnp.transpose(_dot(xr, mpxyz_ref[...]))
    px = pxyz[0:16]
    py = pxyz[16:32]
    pz = pxyz[32:48]
    pt = jnp.sqrt(px ** 2 + py ** 2)
    eta = jnp.transpose(_asinh(pz / pt))
    phi = jnp.transpose(jnp.arctan2(py, px))
    eta0 = _gather_lanes(eta, p0_ref[...], bb)
    eta1 = _gather_lanes(eta, p1_ref[...], bb)
    phi0 = _gather_lanes(phi, p0_ref[...], bb)
    phi1 = _gather_lanes(phi, p1_ref[...], bb)
    deta = eta0 - eta1
    dphi = (phi0 - phi1 + jnp.pi) % (2.0 * jnp.pi) - jnp.pi
    dr = jnp.sqrt(deta ** 2 + dphi ** 2)
    a = jnp.abs(dr - _CONST)
    a = jnp.where(jnp.isnan(a), 3e38, a)
    a = jnp.minimum(a, 3e38)
    m = (_gather_lanes(a, c0_ref[...], bb)
         + _gather_lanes(a, c1_ref[...], bb))
    mn = jnp.min(m, axis=1, keepdims=True)
    li = jnp.broadcast_to(li_ref[...], m.shape)
    idx = jnp.min(jnp.where(m == mn, li, 1e9), axis=1, keepdims=True)
    oh = (li == idx).astype(jnp.float32)
    r = jnp.dot(oh, pay_ref[...], preferred_element_type=jnp.float32)
    labs = r[:, 0:4]
    i0 = r[:, 4:5].astype(jnp.int32)
    i1 = r[:, 5:6].astype(jnp.int32)
    d0 = jnp.take_along_axis(dr, i0, axis=1)
    d1 = jnp.take_along_axis(dr, i1, axis=1)
    o_ref[...] = jnp.concatenate([labs, d0, d1, mn], axis=1)


@jax.jit
def kernel(x):
    b = x.shape[0]
    x2 = x.reshape(b, 4 * _IN_DIM)
    grid = (b // _BB,)
    full = lambda shape: pl.BlockSpec(shape, lambda i: (0, 0))
    out = pl.pallas_call(
        _body,
        grid=grid,
        in_specs=[
            pl.BlockSpec((_BB, 4 * _IN_DIM), lambda i: (i, 0)),
            full(_MPXYZ.shape),
            full(_IP0.shape),
            full(_IP1.shape),
            full(_IC0.shape),
            full(_IC1.shape),
            full(_LIDX.shape),
            full(_PAYLOAD.shape),
        ],
        out_specs=pl.BlockSpec((_BB, 7), lambda i: (i, 0)),
        out_shape=jax.ShapeDtypeStruct((b, 7), jnp.float32),
        compiler_params=pltpu.CompilerParams(
            dimension_semantics=("arbitrary",),
        ),
    )(x2, _MPXYZ, _IP0, _IP1, _IC0, _IC1, _LIDX, _PAYLOAD)
    return out


# speedup vs baseline: 1.0117x; 1.0117x over previous
import functools
import itertools

import numpy as np
import jax
import jax.numpy as jnp
from jax import lax
from jax.experimental import pallas as pl
from jax.experimental.pallas import tpu as pltpu

_IN_DIM = 10
_NCH = 2
_CONST = 0.8
_NP = 45
_NC = 630


def _pair_combos(n, k):
    x = list(set(frozenset(i) for i in itertools.product(range(n), repeat=k)
                 if len(set(i)) == k))
    return np.array(sorted([sorted(list(i)) for i in x]), dtype=np.int64)


def _disjoint_combos(drcombos):
    combos = []
    dc = [set(i) for i in drcombos.tolist()]
    for idx, i in enumerate(dc):
        for jdx, j in enumerate(dc):
            if not i.intersection(j):
                if [idx, jdx] not in combos and [jdx, idx] not in combos:
                    combos.append([idx, jdx])
    return np.array(sorted(combos), dtype=np.int64)


_DRC = _pair_combos(_IN_DIM, _NCH)
_DRSC = _disjoint_combos(_DRC)

_MPXYZ = np.zeros((4 * _IN_DIM, 48), dtype=np.float32)
for _j in range(_IN_DIM):
    for _k, _comp in enumerate((1, 2, 3)):
        _MPXYZ[4 * _j + _comp, 16 * _k + _j] = 1.0

_DMAT = np.zeros((_IN_DIM, _NP), dtype=np.float32)
for _c, (_i, _j) in enumerate(_DRC):
    _DMAT[_i, _c] = 1.0
    _DMAT[_j, _c] = -1.0

_PS = np.zeros((_NP, _NC), dtype=np.float32)
for _c, (_i, _j) in enumerate(_DRSC):
    _PS[_i, _c] = 1.0
    _PS[_j, _c] = 1.0

_PAYLOAD = np.zeros((_NC, 8), dtype=np.float32)
_PAYLOAD[:, 0:4] = _DRC[_DRSC].reshape(_NC, 4).astype(np.float32)
_PAYLOAD[:, 4] = _DRSC[:, 0].astype(np.float32)
_PAYLOAD[:, 5] = _DRSC[:, 1].astype(np.float32)

_IP0 = _DRC[:, 0].astype(np.int32).reshape(1, _NP)
_IP1 = _DRC[:, 1].astype(np.int32).reshape(1, _NP)
_IC0 = _DRSC[:, 0].astype(np.int32).reshape(1, _NC)
_IC1 = _DRSC[:, 1].astype(np.int32).reshape(1, _NC)

_LIDX = np.arange(_NC, dtype=np.float32).reshape(1, _NC)

_BB = 1024


def _dot(a, b):
    return jnp.dot(a, b, precision=lax.Precision.HIGHEST,
                   preferred_element_type=jnp.float32)


def _asinh(t):
    u = jnp.abs(t)
    r = jnp.log1p(u + u * u / (1.0 + jnp.sqrt(u * u + 1.0)))
    r = jnp.where(u > 1e19, jnp.log(u) + 0.6931471805599453, r)
    return jnp.sign(t) * r


def _gather_lanes(src, idx_row, bb):
    idx = jnp.broadcast_to(idx_row, (bb, idx_row.shape[1]))
    return jnp.take_along_axis(src, idx, axis=1)


def _body(x_ref, mpxyz_ref, p0_ref, p1_ref, c0_ref, c1_ref, li_ref,
          pay_ref, o_ref):
    xr = x_ref[...]
    bb = xr.shape[0]
    pxyz = jnp.transpose(_dot(xr, mpxyz_ref[...]))
    px = pxyz[0:16]
    py = pxyz[16:32]
    pz = pxyz[32:48]
    pt = jnp.sqrt(px ** 2 + py ** 2)
    eta = jnp.transpose(_asinh(pz / pt))
    phi = jnp.transpose(jnp.arctan2(py, px))
    eta0 = _gather_lanes(eta, p0_ref[...], bb)
    eta1 = _gather_lanes(eta, p1_ref[...], bb)
    phi0 = _gather_lanes(phi, p0_ref[...], bb)
    phi1 = _gather_lanes(phi, p1_ref[...], bb)
    deta = eta0 - eta1
    dphi = (phi0 - phi1 + jnp.pi) % (2.0 * jnp.pi) - jnp.pi
    dr = jnp.sqrt(deta ** 2 + dphi ** 2)
    a = jnp.abs(dr - _CONST)
    a = jnp.where(jnp.isnan(a), 3e38, a)
    a = jnp.minimum(a, 3e38)
    m = (_gather_lanes(a, c0_ref[...], bb)
         + _gather_lanes(a, c1_ref[...], bb))
    mn = jnp.min(m, axis=1, keepdims=True)
    li = jnp.broadcast_to(li_ref[...], m.shape)
    idx = jnp.min(jnp.where(m == mn, li, 1e9), axis=1, keepdims=True)
    oh = (li == idx).astype(jnp.float32)
    r = jnp.dot(oh, pay_ref[...], preferred_element_type=jnp.float32)
    labs = r[:, 0:4]
    i0 = r[:, 4:5].astype(jnp.int32)
    i1 = r[:, 5:6].astype(jnp.int32)
    d0 = jnp.take_along_axis(dr, i0, axis=1)
    d1 = jnp.take_along_axis(dr, i1, axis=1)
    o_ref[...] = jnp.concatenate([labs, d0, d1, mn], axis=1)


@jax.jit
def kernel(x):
    b = x.shape[0]
    x2 = x.reshape(b, 4 * _IN_DIM)
    grid = (b // _BB,)
    full = lambda shape: pl.BlockSpec(shape, lambda i: (0, 0))
    out = pl.pallas_call(
        _body,
        grid=grid,
        in_specs=[
            pl.BlockSpec((_BB, 4 * _IN_DIM), lambda i: (i, 0)),
            full(_MPXYZ.shape),
            full(_IP0.shape),
            full(_IP1.shape),
            full(_IC0.shape),
            full(_IC1.shape),
            full(_LIDX.shape),
            full(_PAYLOAD.shape),
        ],
        out_specs=pl.BlockSpec((_BB, 7), lambda i: (i, 0)),
        out_shape=jax.ShapeDtypeStruct((b, 7), jnp.float32),
        compiler_params=pltpu.CompilerParams(
            dimension_semantics=("arbitrary",),
        ),
    )(x2, _MPXYZ, _IP0, _IP1, _IC0, _IC1, _LIDX, _PAYLOAD)
    return out


# speedup vs baseline: 1.0767x; 1.0643x over previous
import functools
import itertools

import numpy as np
import jax
import jax.numpy as jnp
from jax import lax
from jax.experimental import pallas as pl
from jax.experimental.pallas import tpu as pltpu

_IN_DIM = 10
_NCH = 2
_CONST = 0.8
_NP = 45
_NC = 630


def _pair_combos(n, k):
    x = list(set(frozenset(i) for i in itertools.product(range(n), repeat=k)
                 if len(set(i)) == k))
    return np.array(sorted([sorted(list(i)) for i in x]), dtype=np.int64)


def _disjoint_combos(drcombos):
    combos = []
    dc = [set(i) for i in drcombos.tolist()]
    for idx, i in enumerate(dc):
        for jdx, j in enumerate(dc):
            if not i.intersection(j):
                if [idx, jdx] not in combos and [jdx, idx] not in combos:
                    combos.append([idx, jdx])
    return np.array(sorted(combos), dtype=np.int64)


_DRC = _pair_combos(_IN_DIM, _NCH)
_DRSC = _disjoint_combos(_DRC)

_MPXYZ = np.zeros((4 * _IN_DIM, 48), dtype=np.float32)
for _j in range(_IN_DIM):
    for _k, _comp in enumerate((1, 2, 3)):
        _MPXYZ[4 * _j + _comp, 16 * _k + _j] = 1.0

_DMAT = np.zeros((_IN_DIM, _NP), dtype=np.float32)
for _c, (_i, _j) in enumerate(_DRC):
    _DMAT[_i, _c] = 1.0
    _DMAT[_j, _c] = -1.0

_PS = np.zeros((_NP, _NC), dtype=np.float32)
for _c, (_i, _j) in enumerate(_DRSC):
    _PS[_i, _c] = 1.0
    _PS[_j, _c] = 1.0

_PAYLOAD = np.zeros((640, 8), dtype=np.float32)
_PAYLOAD[:_NC, 0:4] = _DRC[_DRSC].reshape(_NC, 4).astype(np.float32)
_PAYLOAD[:_NC, 4] = _DRSC[:, 0].astype(np.float32)
_PAYLOAD[:_NC, 5] = _DRSC[:, 1].astype(np.float32)

_NCP = 640
_IP0 = _DRC[:, 0].astype(np.int32).reshape(1, _NP)
_IP1 = _DRC[:, 1].astype(np.int32).reshape(1, _NP)
_IC0 = np.full((1, _NCP), _DRSC[-1, 0], dtype=np.int32)
_IC0[0, :_NC] = _DRSC[:, 0]
_IC1 = np.full((1, _NCP), _DRSC[-1, 1], dtype=np.int32)
_IC1[0, :_NC] = _DRSC[:, 1]

_LIDX = np.full((1, _NCP), _NC - 1, dtype=np.float32)
_LIDX[0, :_NC] = np.arange(_NC, dtype=np.float32)

_BB = 1024
_RC = 256
_CT = 128


def _dot(a, b):
    return jnp.dot(a, b, precision=lax.Precision.HIGHEST,
                   preferred_element_type=jnp.float32)


def _asinh(t):
    u = jnp.abs(t)
    r = jnp.log1p(u + u * u / (1.0 + jnp.sqrt(u * u + 1.0)))
    r = jnp.where(u > 1e19, jnp.log(u) + 0.6931471805599453, r)
    return jnp.sign(t) * r


def _gather_lanes(src, idx_row, bb):
    idx = jnp.broadcast_to(idx_row, (bb, idx_row.shape[1]))
    return jnp.take_along_axis(src, idx, axis=1)


def _pipeline(xr, mpxyz, p0, p1, c0, c1, li_row, pay):
    rc = xr.shape[0]
    pxyz = jnp.transpose(_dot(xr, mpxyz))
    px = pxyz[0:16]
    py = pxyz[16:32]
    pz = pxyz[32:48]
    pt = jnp.sqrt(px ** 2 + py ** 2)
    eta = jnp.transpose(_asinh(pz / pt))
    phi = jnp.transpose(jnp.arctan2(py, px))
    eta0 = _gather_lanes(eta, p0, rc)
    eta1 = _gather_lanes(eta, p1, rc)
    phi0 = _gather_lanes(phi, p0, rc)
    phi1 = _gather_lanes(phi, p1, rc)
    deta = eta0 - eta1
    dphi = (phi0 - phi1 + jnp.pi) % (2.0 * jnp.pi) - jnp.pi
    dr = jnp.sqrt(deta ** 2 + dphi ** 2)
    a = jnp.abs(dr - _CONST)
    a = jnp.where(jnp.isnan(a), 3e38, a)
    a = jnp.minimum(a, 3e38)
    best_v = jnp.full((rc, _CT), jnp.inf, dtype=jnp.float32)
    best_i = jnp.zeros((rc, _CT), dtype=jnp.float32)
    for t in range(_NCP // _CT):
        cols = slice(t * _CT, (t + 1) * _CT)
        g = (_gather_lanes(a, c0[:, cols], rc)
             + _gather_lanes(a, c1[:, cols], rc))
        upd = g < best_v
        best_v = jnp.minimum(best_v, g)
        best_i = jnp.where(upd, jnp.broadcast_to(li_row[:, cols], g.shape),
                           best_i)
    mn = jnp.min(best_v, axis=1, keepdims=True)
    idx = jnp.min(jnp.where(best_v == mn, best_i, 1e9),
                  axis=1, keepdims=True)
    r = None
    for t in range(_NCP // _CT):
        cols = slice(t * _CT, (t + 1) * _CT)
        oh_t = (jnp.broadcast_to(li_row[:, cols], (rc, _CT))
                == idx).astype(jnp.float32)
        rt = jnp.dot(oh_t, pay[cols, :], preferred_element_type=jnp.float32)
        r = rt if r is None else r + rt
    labs = r[:, 0:4]
    i0 = r[:, 4:5].astype(jnp.int32)
    i1 = r[:, 5:6].astype(jnp.int32)
    d0 = jnp.take_along_axis(dr, i0, axis=1)
    d1 = jnp.take_along_axis(dr, i1, axis=1)
    return labs, d0, d1, mn


def _body(x_ref, mpxyz_ref, p0_ref, p1_ref, c0_ref, c1_ref, li_ref,
          pay_ref, o_ref):
    mpxyz = mpxyz_ref[...]
    p0 = p0_ref[...]
    p1 = p1_ref[...]
    c0 = c0_ref[...]
    c1 = c1_ref[...]
    li_row = li_ref[...]
    pay = pay_ref[...]
    for h in range(_BB // _RC):
        rows = pl.ds(h * _RC, _RC)
        xr = x_ref[rows, :]
        labs, d0, d1, mn = _pipeline(xr, mpxyz, p0, p1, c0, c1, li_row, pay)
        o_ref[rows, 0:4] = labs
        o_ref[rows, 4:5] = d0
        o_ref[rows, 5:6] = d1
        o_ref[rows, 6:7] = mn


@jax.jit
def kernel(x):
    b = x.shape[0]
    x2 = x.reshape(b, 4 * _IN_DIM)
    grid = (b // _BB,)
    full = lambda shape: pl.BlockSpec(shape, lambda i: (0, 0))
    out = pl.pallas_call(
        _body,
        grid=grid,
        in_specs=[
            pl.BlockSpec((_BB, 4 * _IN_DIM), lambda i: (i, 0)),
            full(_MPXYZ.shape),
            full(_IP0.shape),
            full(_IP1.shape),
            full(_IC0.shape),
            full(_IC1.shape),
            full(_LIDX.shape),
            full(_PAYLOAD.shape),
        ],
        out_specs=pl.BlockSpec((_BB, 7), lambda i: (i, 0)),
        out_shape=jax.ShapeDtypeStruct((b, 7), jnp.float32),
        compiler_params=pltpu.CompilerParams(
            dimension_semantics=("arbitrary",),
        ),
    )(x2, _MPXYZ, _IP0, _IP1, _IC0, _IC1, _LIDX, _PAYLOAD)
    return out


# speedup vs baseline: 1.1170x; 1.0374x over previous
import functools
import itertools

import numpy as np
import jax
import jax.numpy as jnp
from jax import lax
from jax.experimental import pallas as pl
from jax.experimental.pallas import tpu as pltpu

_IN_DIM = 10
_NCH = 2
_CONST = 0.8
_NP = 45
_NC = 630


def _pair_combos(n, k):
    x = list(set(frozenset(i) for i in itertools.product(range(n), repeat=k)
                 if len(set(i)) == k))
    return np.array(sorted([sorted(list(i)) for i in x]), dtype=np.int64)


def _disjoint_combos(drcombos):
    combos = []
    dc = [set(i) for i in drcombos.tolist()]
    for idx, i in enumerate(dc):
        for jdx, j in enumerate(dc):
            if not i.intersection(j):
                if [idx, jdx] not in combos and [jdx, idx] not in combos:
                    combos.append([idx, jdx])
    return np.array(sorted(combos), dtype=np.int64)


_DRC = _pair_combos(_IN_DIM, _NCH)
_DRSC = _disjoint_combos(_DRC)

_MPXYZ = np.zeros((4 * _IN_DIM, 48), dtype=np.float32)
for _j in range(_IN_DIM):
    for _k, _comp in enumerate((1, 2, 3)):
        _MPXYZ[4 * _j + _comp, 16 * _k + _j] = 1.0

_DMT = np.zeros((48, 16), dtype=np.float32)
for _c, (_i, _j) in enumerate(_DRC):
    _DMT[_c, _i] = 1.0
    _DMT[_c, _j] = -1.0

_PS = np.zeros((_NP, _NC), dtype=np.float32)
for _c, (_i, _j) in enumerate(_DRSC):
    _PS[_i, _c] = 1.0
    _PS[_j, _c] = 1.0

_PAYLOAD = np.zeros((640, 8), dtype=np.float32)
_PAYLOAD[:_NC, 0:4] = _DRC[_DRSC].reshape(_NC, 4).astype(np.float32)
_PAYLOAD[:_NC, 4] = _DRSC[:, 0].astype(np.float32)
_PAYLOAD[:_NC, 5] = _DRSC[:, 1].astype(np.float32)

_NCP = 640
_IP0 = _DRC[:, 0].astype(np.int32).reshape(1, _NP)
_IP1 = _DRC[:, 1].astype(np.int32).reshape(1, _NP)
_IC0 = np.full((1, _NCP), _DRSC[-1, 0], dtype=np.int32)
_IC0[0, :_NC] = _DRSC[:, 0]
_IC1 = np.full((1, _NCP), _DRSC[-1, 1], dtype=np.int32)
_IC1[0, :_NC] = _DRSC[:, 1]

_LIDX = np.full((1, _NCP), _NC - 1, dtype=np.float32)
_LIDX[0, :_NC] = np.arange(_NC, dtype=np.float32)

_BB = 1024
_RC = 256
_CT = 128


def _dot(a, b):
    return jnp.dot(a, b, precision=lax.Precision.HIGHEST,
                   preferred_element_type=jnp.float32)


def _asinh(t):
    u = jnp.abs(t)
    r = jnp.log1p(u + u * u / (1.0 + jnp.sqrt(u * u + 1.0)))
    r = jnp.where(u > 1e19, jnp.log(u) + 0.6931471805599453, r)
    return jnp.sign(t) * r


def _gather_lanes(src, idx_row, bb):
    idx = jnp.broadcast_to(idx_row, (bb, idx_row.shape[1]))
    return jnp.take_along_axis(src, idx, axis=1)


def _pipeline(xr, mpxyz, dmt, c0, c1, li_row, pay):
    rc = xr.shape[0]
    pxyz = jnp.transpose(_dot(xr, mpxyz))
    px = pxyz[0:16]
    py = pxyz[16:32]
    pz = pxyz[32:48]
    pt = jnp.sqrt(px ** 2 + py ** 2)
    eta = _asinh(pz / pt)
    phi = jnp.arctan2(py, px)
    rmask = lax.broadcasted_iota(jnp.int32, eta.shape, 0) < _IN_DIM
    eta = jnp.where(rmask, eta, 0.0)
    deta = _dot(dmt, eta)
    dphi = (_dot(dmt, phi) + jnp.pi) % (2.0 * jnp.pi) - jnp.pi
    drt = jnp.sqrt(deta ** 2 + dphi ** 2)
    at = jnp.abs(drt - _CONST)
    at = jnp.where(jnp.isnan(at), 3e38, at)
    at = jnp.minimum(at, 3e38)
    a = jnp.transpose(at)
    dr = jnp.transpose(drt)
    best_v = jnp.full((rc, _CT), jnp.inf, dtype=jnp.float32)
    best_i = jnp.zeros((rc, _CT), dtype=jnp.float32)
    for t in range(_NCP // _CT):
        cols = slice(t * _CT, (t + 1) * _CT)
        g = (_gather_lanes(a, c0[:, cols], rc)
             + _gather_lanes(a, c1[:, cols], rc))
        upd = g < best_v
        best_v = jnp.minimum(best_v, g)
        best_i = jnp.where(upd, jnp.broadcast_to(li_row[:, cols], g.shape),
                           best_i)
    mn = jnp.min(best_v, axis=1, keepdims=True)
    idx = jnp.min(jnp.where(best_v == mn, best_i, 1e9),
                  axis=1, keepdims=True)
    r = None
    for t in range(_NCP // _CT):
        cols = slice(t * _CT, (t + 1) * _CT)
        oh_t = (jnp.broadcast_to(li_row[:, cols], (rc, _CT))
                == idx).astype(jnp.float32)
        rt = jnp.dot(oh_t, pay[cols, :], preferred_element_type=jnp.float32)
        r = rt if r is None else r + rt
    labs = r[:, 0:4]
    i0 = r[:, 4:5].astype(jnp.int32)
    i1 = r[:, 5:6].astype(jnp.int32)
    d0 = jnp.take_along_axis(dr, i0, axis=1)
    d1 = jnp.take_along_axis(dr, i1, axis=1)
    return labs, d0, d1, mn


def _body(x_ref, mpxyz_ref, dmt_ref, c0_ref, c1_ref, li_ref,
          pay_ref, o_ref):
    mpxyz = mpxyz_ref[...]
    dmt = dmt_ref[...]
    c0 = c0_ref[...]
    c1 = c1_ref[...]
    li_row = li_ref[...]
    pay = pay_ref[...]
    for h in range(_BB // _RC):
        rows = pl.ds(h * _RC, _RC)
        xr = x_ref[rows, :]
        labs, d0, d1, mn = _pipeline(xr, mpxyz, dmt, c0, c1, li_row, pay)
        o_ref[rows, 0:4] = labs
        o_ref[rows, 4:5] = d0
        o_ref[rows, 5:6] = d1
        o_ref[rows, 6:7] = mn


@jax.jit
def kernel(x):
    b = x.shape[0]
    x2 = x.reshape(b, 4 * _IN_DIM)
    grid = (b // _BB,)
    full = lambda shape: pl.BlockSpec(shape, lambda i: (0, 0))
    out = pl.pallas_call(
        _body,
        grid=grid,
        in_specs=[
            pl.BlockSpec((_BB, 4 * _IN_DIM), lambda i: (i, 0)),
            full(_MPXYZ.shape),
            full(_DMT.shape),
            full(_IC0.shape),
            full(_IC1.shape),
            full(_LIDX.shape),
            full(_PAYLOAD.shape),
        ],
        out_specs=pl.BlockSpec((_BB, 7), lambda i: (i, 0)),
        out_shape=jax.ShapeDtypeStruct((b, 7), jnp.float32),
        compiler_params=pltpu.CompilerParams(
            dimension_semantics=("arbitrary",),
        ),
    )(x2, _MPXYZ, _DMT, _IC0, _IC1, _LIDX, _PAYLOAD)
    return out
